# initial kernel scaffold (unmeasured)
import jax
import jax.numpy as jnp
from jax import lax
from jax.experimental import pallas as pl
from jax.experimental.pallas import tpu as pltpu

N_DEV = 4


def kernel(x, w_mat):
    m, _ = x.shape
    _, n = w_mat.shape
    mc = m // N_DEV

    partial = jnp.dot(
        x.astype(jnp.bfloat16),
        w_mat.astype(jnp.bfloat16),
        preferred_element_type=jnp.float32,
    )

    def body(p_ref, out_ref, stage_ref, rs_recv_ref,
             rs_send_sems, rs_recv_sems, ag_send_sems, ag_recv_sems):
        del p_ref
        my = lax.axis_index("i")
        left = lax.rem(my + (N_DEV - 1), N_DEV)
        right = lax.rem(my + 1, N_DEV)

        barrier = pltpu.get_barrier_semaphore()
        for nbr in (left, right):
            pl.semaphore_signal(
                barrier, inc=1, device_id=(nbr,),
                device_id_type=pl.DeviceIdType.MESH,
            )
        pl.semaphore_wait(barrier, 2)

        for h in range(N_DEV - 1):
            s = lax.rem(my + (N_DEV - h), N_DEV)
            stage_ref[...] = out_ref[pl.ds(s * mc, mc), :].astype(jnp.bfloat16)
            rdma = pltpu.make_async_remote_copy(
                src_ref=stage_ref,
                dst_ref=rs_recv_ref.at[h],
                send_sem=rs_send_sems.at[h],
                recv_sem=rs_recv_sems.at[h],
                device_id=(right,),
                device_id_type=pl.DeviceIdType.MESH,
            )
            rdma.start()
            rdma.wait()
            r = lax.rem(my + (2 * N_DEV - h - 1), N_DEV)
            sl = pl.ds(r * mc, mc)
            out_ref[sl, :] = out_ref[sl, :] + rs_recv_ref[h].astype(jnp.float32)

        for h in range(N_DEV - 1):
            g = lax.rem(my + (N_DEV + 1 - h), N_DEV)
            sl = pl.ds(g * mc, mc)
            rdma = pltpu.make_async_remote_copy(
                src_ref=out_ref.at[sl, :],
                dst_ref=out_ref.at[sl, :],
                send_sem=ag_send_sems.at[h],
                recv_sem=ag_recv_sems.at[h],
                device_id=(right,),
                device_id_type=pl.DeviceIdType.MESH,
            )
            rdma.start()
            rdma.wait()

    return pl.pallas_call(
        body,
        out_shape=jax.ShapeDtypeStruct((m, n), jnp.float32),
        in_specs=[pl.BlockSpec(memory_space=pltpu.VMEM)],
        out_specs=pl.BlockSpec(memory_space=pltpu.VMEM),
        scratch_shapes=[
            pltpu.VMEM((mc, n), jnp.bfloat16),
            pltpu.VMEM((N_DEV - 1, mc, n), jnp.bfloat16),
            pltpu.SemaphoreType.DMA((N_DEV - 1,)),
            pltpu.SemaphoreType.DMA((N_DEV - 1,)),
            pltpu.SemaphoreType.DMA((N_DEV - 1,)),
            pltpu.SemaphoreType.DMA((N_DEV - 1,)),
        ],
        input_output_aliases={0: 0},
        compiler_params=pltpu.CompilerParams(
            collective_id=0,
            vmem_limit_bytes=64 * 1024 * 1024,
        ),
    )(partial)


# baseline (device time: 492162 ns/iter reference)
import jax
import jax.numpy as jnp
from jax import lax
from jax.experimental import pallas as pl
from jax.experimental.pallas import tpu as pltpu

N_DEV = 4


def kernel(x, w_mat):
    m, _ = x.shape
    _, n = w_mat.shape
    mc = m // N_DEV

    partial = jnp.dot(
        x.astype(jnp.bfloat16),
        w_mat.astype(jnp.bfloat16),
        preferred_element_type=jnp.float32,
    )

    def body(p_ref, out_ref, stage_ref, rs_recv_ref,
             rs_send_sems, rs_recv_sems, ag_send_sems, ag_recv_sems, copy_sem):
        my = lax.axis_index("i")
        left = lax.rem(my + (N_DEV - 1), N_DEV)
        right = lax.rem(my + 1, N_DEV)

        copy = pltpu.make_async_copy(p_ref, out_ref, copy_sem)
        copy.start()

        barrier = pltpu.get_barrier_semaphore()
        for nbr in (left, right):
            pl.semaphore_signal(
                barrier, inc=1, device_id=(nbr,),
                device_id_type=pl.DeviceIdType.MESH,
            )
        pl.semaphore_wait(barrier, 2)
        copy.wait()

        for h in range(N_DEV - 1):
            s = lax.rem(my + (N_DEV - h), N_DEV)
            stage_ref[...] = out_ref[pl.ds(s * mc, mc), :].astype(jnp.bfloat16)
            rdma = pltpu.make_async_remote_copy(
                src_ref=stage_ref,
                dst_ref=rs_recv_ref.at[h],
                send_sem=rs_send_sems.at[h],
                recv_sem=rs_recv_sems.at[h],
                device_id=(right,),
                device_id_type=pl.DeviceIdType.MESH,
            )
            rdma.start()
            rdma.wait()
            r = lax.rem(my + (2 * N_DEV - h - 1), N_DEV)
            sl = pl.ds(r * mc, mc)
            out_ref[sl, :] = out_ref[sl, :] + rs_recv_ref[h].astype(jnp.float32)

        for h in range(N_DEV - 1):
            g = lax.rem(my + (N_DEV + 1 - h), N_DEV)
            sl = pl.ds(g * mc, mc)
            rdma = pltpu.make_async_remote_copy(
                src_ref=out_ref.at[sl, :],
                dst_ref=out_ref.at[sl, :],
                send_sem=ag_send_sems.at[h],
                recv_sem=ag_recv_sems.at[h],
                device_id=(right,),
                device_id_type=pl.DeviceIdType.MESH,
            )
            rdma.start()
            rdma.wait()

    return pl.pallas_call(
        body,
        out_shape=jax.ShapeDtypeStruct((m, n), jnp.float32),
        in_specs=[pl.BlockSpec(memory_space=pl.ANY)],
        out_specs=pl.BlockSpec(memory_space=pltpu.VMEM),
        scratch_shapes=[
            pltpu.VMEM((mc, n), jnp.bfloat16),
            pltpu.VMEM((N_DEV - 1, mc, n), jnp.bfloat16),
            pltpu.SemaphoreType.DMA((N_DEV - 1,)),
            pltpu.SemaphoreType.DMA((N_DEV - 1,)),
            pltpu.SemaphoreType.DMA((N_DEV - 1,)),
            pltpu.SemaphoreType.DMA((N_DEV - 1,)),
            pltpu.SemaphoreType.DMA(()),
        ],
        compiler_params=pltpu.CompilerParams(
            collective_id=0,
            vmem_limit_bytes=64 * 1024 * 1024,
        ),
    )(partial)


# device time: 223485 ns/iter; 2.2022x vs baseline; 2.2022x over previous
import jax
import jax.numpy as jnp
from jax import lax
from jax.experimental import pallas as pl
from jax.experimental.pallas import tpu as pltpu

N_DEV = 4
MESH = pl.DeviceIdType.MESH


def kernel(x, w_mat):
    m, _ = x.shape
    _, n = w_mat.shape
    mc = m // N_DEV
    nh = n // 2

    partial = jnp.dot(
        x.astype(jnp.bfloat16),
        w_mat.astype(jnp.bfloat16),
        preferred_element_type=jnp.float32,
    )

    def body(p_ref, out_ref, stage_p, stage_m, slot_p, slot_m,
             ag_slot_p, ag_slot_m,
             rs_send_p, rs_recv_p, rs_send_m, rs_recv_m,
             ag_send_p, ag_recv_p, ag_send_m, ag_recv_m,
             copy_sem):
        my = lax.axis_index("i")
        left = lax.rem(my + (N_DEV - 1), N_DEV)
        right = lax.rem(my + 1, N_DEV)

        copy = pltpu.make_async_copy(p_ref, out_ref, copy_sem)
        copy.start()

        barrier = pltpu.get_barrier_semaphore()
        for nbr in (left, right):
            pl.semaphore_signal(
                barrier, inc=1, device_id=(nbr,), device_id_type=MESH,
            )
        pl.semaphore_wait(barrier, 2)
        copy.wait()

        def rows(c):
            return pl.ds(c * mc, mc)

        L = slice(0, nh)
        R = slice(nh, n)

        stage_p[...] = out_ref[rows(my), L].astype(jnp.bfloat16)
        stage_m[...] = out_ref[rows(my), R].astype(jnp.bfloat16)
        for h in range(N_DEV - 1):
            rp = pltpu.make_async_remote_copy(
                src_ref=stage_p, dst_ref=slot_p.at[h],
                send_sem=rs_send_p.at[h], recv_sem=rs_recv_p.at[h],
                device_id=(right,), device_id_type=MESH)
            rm = pltpu.make_async_remote_copy(
                src_ref=stage_m, dst_ref=slot_m.at[h],
                send_sem=rs_send_m.at[h], recv_sem=rs_recv_m.at[h],
                device_id=(left,), device_id_type=MESH)
            rp.start()
            rm.start()

            cp = lax.rem(my + (2 * N_DEV - h - 1), N_DEV)
            cm = lax.rem(my + h + 1, N_DEV)

            rp.wait()
            accp = out_ref[rows(cp), L] + slot_p[h].astype(jnp.float32)
            out_ref[rows(cp), L] = accp
            if h < N_DEV - 2:
                stage_p[...] = accp.astype(jnp.bfloat16)

            rm.wait()
            accm = out_ref[rows(cm), R] + slot_m[h].astype(jnp.float32)
            out_ref[rows(cm), R] = accm
            if h < N_DEV - 2:
                stage_m[...] = accm.astype(jnp.bfloat16)

        stage_p[...] = out_ref[rows(lax.rem(my + 1, N_DEV)), L].astype(jnp.bfloat16)
        stage_m[...] = out_ref[rows(lax.rem(my + (N_DEV - 1), N_DEV)), R].astype(jnp.bfloat16)

        descs = []
        for h in range(N_DEV - 1):
            ap = pltpu.make_async_remote_copy(
                src_ref=stage_p if h == 0 else ag_slot_p.at[h - 1],
                dst_ref=ag_slot_p.at[h],
                send_sem=ag_send_p.at[h], recv_sem=ag_recv_p.at[h],
                device_id=(right,), device_id_type=MESH)
            am = pltpu.make_async_remote_copy(
                src_ref=stage_m if h == 0 else ag_slot_m.at[h - 1],
                dst_ref=ag_slot_m.at[h],
                send_sem=ag_send_m.at[h], recv_sem=ag_recv_m.at[h],
                device_id=(left,), device_id_type=MESH)
            if h > 0:
                descs[h - 1][0].wait_recv()
            ap.start()
            if h > 0:
                descs[h - 1][1].wait_recv()
            am.start()
            if h > 0:
                out_ref[rows(lax.rem(my + (N_DEV - h + 1), N_DEV)), L] = (
                    ag_slot_p[h - 1].astype(jnp.float32))
                out_ref[rows(lax.rem(my + h - 1, N_DEV)), R] = (
                    ag_slot_m[h - 1].astype(jnp.float32))
            descs.append((ap, am))

        descs[-1][0].wait_recv()
        descs[-1][1].wait_recv()
        h = N_DEV - 1
        out_ref[rows(lax.rem(my + (N_DEV - h + 1), N_DEV)), L] = (
            ag_slot_p[h - 1].astype(jnp.float32))
        out_ref[rows(lax.rem(my + h - 1, N_DEV)), R] = (
            ag_slot_m[h - 1].astype(jnp.float32))
        for ap, am in descs:
            ap.wait_send()
            am.wait_send()

    return pl.pallas_call(
        body,
        out_shape=jax.ShapeDtypeStruct((m, n), jnp.float32),
        in_specs=[pl.BlockSpec(memory_space=pl.ANY)],
        out_specs=pl.BlockSpec(memory_space=pltpu.VMEM),
        scratch_shapes=[
            pltpu.VMEM((mc, nh), jnp.bfloat16),
            pltpu.VMEM((mc, nh), jnp.bfloat16),
            pltpu.VMEM((N_DEV - 1, mc, nh), jnp.bfloat16),
            pltpu.VMEM((N_DEV - 1, mc, nh), jnp.bfloat16),
            pltpu.VMEM((N_DEV - 1, mc, nh), jnp.bfloat16),
            pltpu.VMEM((N_DEV - 1, mc, nh), jnp.bfloat16),
            pltpu.SemaphoreType.DMA((N_DEV - 1,)),
            pltpu.SemaphoreType.DMA((N_DEV - 1,)),
            pltpu.SemaphoreType.DMA((N_DEV - 1,)),
            pltpu.SemaphoreType.DMA((N_DEV - 1,)),
            pltpu.SemaphoreType.DMA((N_DEV - 1,)),
            pltpu.SemaphoreType.DMA((N_DEV - 1,)),
            pltpu.SemaphoreType.DMA((N_DEV - 1,)),
            pltpu.SemaphoreType.DMA((N_DEV - 1,)),
            pltpu.SemaphoreType.DMA(()),
        ],
        compiler_params=pltpu.CompilerParams(
            collective_id=0,
            vmem_limit_bytes=64 * 1024 * 1024,
        ),
    )(partial)


# device time: 208515 ns/iter; 2.3603x vs baseline; 1.0718x over previous
import jax
import jax.numpy as jnp
from jax import lax
from jax.experimental import pallas as pl
from jax.experimental.pallas import tpu as pltpu

N_DEV = 4
MESH = pl.DeviceIdType.MESH


def kernel(x, w_mat):
    m, _ = x.shape
    _, n = w_mat.shape
    mc = m // N_DEV
    nh = n // 2

    partial = jnp.dot(
        x.astype(jnp.bfloat16),
        w_mat.astype(jnp.bfloat16),
        preferred_element_type=jnp.float32,
    ).astype(jnp.bfloat16)

    def body(p_ref, out_ref, result, slot_p, slot_m, fstage,
             rs_send_p, rs_recv_p, rs_send_m, rs_recv_m,
             ag_send_p, ag_recv_p, ag_send_m, ag_recv_m,
             flush_sems, copy_sem):
        my = lax.axis_index("i")
        left = lax.rem(my + (N_DEV - 1), N_DEV)
        right = lax.rem(my + 1, N_DEV)

        copy_in = pltpu.make_async_copy(p_ref, result, copy_sem)
        copy_in.start()

        barrier = pltpu.get_barrier_semaphore()
        for nbr in (left, right):
            pl.semaphore_signal(
                barrier, inc=1, device_id=(nbr,), device_id_type=MESH,
            )
        pl.semaphore_wait(barrier, 2)
        copy_in.wait()

        def rows(c):
            return pl.ds(lax.rem(c + 4 * N_DEV, N_DEV) * mc, mc)

        L = slice(0, nh)
        R = slice(nh, n)
        f32 = jnp.float32
        bf16 = jnp.bfloat16

        for h in range(N_DEV - 1):
            rp = pltpu.make_async_remote_copy(
                src_ref=result.at[rows(my - h), L], dst_ref=slot_p.at[h],
                send_sem=rs_send_p.at[h], recv_sem=rs_recv_p.at[h],
                device_id=(right,), device_id_type=MESH)
            rm = pltpu.make_async_remote_copy(
                src_ref=result.at[rows(my + h), R], dst_ref=slot_m.at[h],
                send_sem=rs_send_m.at[h], recv_sem=rs_recv_m.at[h],
                device_id=(left,), device_id_type=MESH)
            rp.start()
            rm.start()
            rp.wait()
            slp = rows(my - h - 1)
            result[slp, L] = (
                result[slp, L].astype(f32) + slot_p[h].astype(f32)
            ).astype(bf16)
            rm.wait()
            slm = rows(my + h + 1)
            result[slm, R] = (
                result[slm, R].astype(f32) + slot_m[h].astype(f32)
            ).astype(bf16)

        pending = [None, None]

        def flush(idx, c, half):
            s = idx % 2
            if pending[s] is not None:
                pending[s].wait()
            fstage[s] = result[rows(c), half].astype(f32)
            d = pltpu.make_async_copy(
                fstage.at[s], out_ref.at[rows(c), half], flush_sems.at[s])
            d.start()
            pending[s] = d

        descs = []
        for h in range(N_DEV - 1):
            ap = pltpu.make_async_remote_copy(
                src_ref=result.at[rows(my + 1 - h), L],
                dst_ref=result.at[rows(my + 1 - h), L],
                send_sem=ag_send_p.at[h], recv_sem=ag_recv_p.at[h],
                device_id=(right,), device_id_type=MESH)
            am = pltpu.make_async_remote_copy(
                src_ref=result.at[rows(my - 1 + h), R],
                dst_ref=result.at[rows(my - 1 + h), R],
                send_sem=ag_send_m.at[h], recv_sem=ag_recv_m.at[h],
                device_id=(left,), device_id_type=MESH)
            if h > 0:
                descs[h - 1][0].wait_recv()
                descs[h - 1][1].wait_recv()
            ap.start()
            am.start()
            if h == 0:
                flush(0, my + 1, L)
                flush(1, my - 1, R)
            else:
                flush(2 * h, my - (h - 1), L)
                flush(2 * h + 1, my + (h - 1), R)
            descs.append((ap, am))

        descs[-1][0].wait_recv()
        descs[-1][1].wait_recv()
        flush(6, my - 2, L)
        flush(7, my + 2, R)
        pending[0].wait()
        pending[1].wait()
        for ap, am in descs:
            ap.wait_send()
            am.wait_send()

    return pl.pallas_call(
        body,
        out_shape=jax.ShapeDtypeStruct((m, n), jnp.float32),
        in_specs=[pl.BlockSpec(memory_space=pl.ANY)],
        out_specs=pl.BlockSpec(memory_space=pl.ANY),
        scratch_shapes=[
            pltpu.VMEM((m, n), jnp.bfloat16),
            pltpu.VMEM((N_DEV - 1, mc, nh), jnp.bfloat16),
            pltpu.VMEM((N_DEV - 1, mc, nh), jnp.bfloat16),
            pltpu.VMEM((2, mc, nh), jnp.float32),
            pltpu.SemaphoreType.DMA((N_DEV - 1,)),
            pltpu.SemaphoreType.DMA((N_DEV - 1,)),
            pltpu.SemaphoreType.DMA((N_DEV - 1,)),
            pltpu.SemaphoreType.DMA((N_DEV - 1,)),
            pltpu.SemaphoreType.DMA((N_DEV - 1,)),
            pltpu.SemaphoreType.DMA((N_DEV - 1,)),
            pltpu.SemaphoreType.DMA((N_DEV - 1,)),
            pltpu.SemaphoreType.DMA((N_DEV - 1,)),
            pltpu.SemaphoreType.DMA((2,)),
            pltpu.SemaphoreType.DMA(()),
        ],
        compiler_params=pltpu.CompilerParams(
            collective_id=0,
            vmem_limit_bytes=64 * 1024 * 1024,
        ),
    )(partial)


# device time: 193922 ns/iter; 2.5379x vs baseline; 1.0753x over previous
import jax
import jax.numpy as jnp
from jax import lax
from jax.experimental import pallas as pl
from jax.experimental.pallas import tpu as pltpu

N_DEV = 4
MESH = pl.DeviceIdType.MESH


def kernel(x, w_mat):
    m, _ = x.shape
    _, n = w_mat.shape
    mc = m // N_DEV
    nh = n // 2

    w_bf16 = w_mat.astype(jnp.bfloat16)

    def body(x_ref, w_ref, out_ref, result, slot_p, slot_m, fstage,
             rs_send_p, rs_recv_p, rs_send_m, rs_recv_m,
             ag_send_p, ag_recv_p, ag_send_m, ag_recv_m,
             flush_sems):
        my = lax.axis_index("i")
        left = lax.rem(my + (N_DEV - 1), N_DEV)
        right = lax.rem(my + 1, N_DEV)

        barrier = pltpu.get_barrier_semaphore()
        for nbr in (left, right):
            pl.semaphore_signal(
                barrier, inc=1, device_id=(nbr,), device_id_type=MESH,
            )

        def rows(c):
            return pl.ds(lax.rem(c + 4 * N_DEV, N_DEV) * mc, mc)

        L = slice(0, nh)
        R = slice(nh, n)
        f32 = jnp.float32
        bf16 = jnp.bfloat16

        def gemm(rsl, csl):
            result[rsl, csl] = jnp.dot(
                x_ref[rsl, :].astype(bf16), w_ref[:, csl],
                preferred_element_type=f32,
            ).astype(bf16)

        gemm(rows(my), L)
        gemm(rows(my), R)
        pl.semaphore_wait(barrier, 2)

        for h in range(N_DEV - 1):
            rp = pltpu.make_async_remote_copy(
                src_ref=result.at[rows(my - h), L], dst_ref=slot_p.at[h],
                send_sem=rs_send_p.at[h], recv_sem=rs_recv_p.at[h],
                device_id=(right,), device_id_type=MESH)
            rm = pltpu.make_async_remote_copy(
                src_ref=result.at[rows(my + h), R], dst_ref=slot_m.at[h],
                send_sem=rs_send_m.at[h], recv_sem=rs_recv_m.at[h],
                device_id=(left,), device_id_type=MESH)
            rp.start()
            rm.start()
            if h == 0:
                gemm(rows(my + 1), slice(0, n))
                gemm(rows(my - 1), slice(0, n))
                gemm(rows(my + 2), slice(0, n))
            rp.wait()
            slp = rows(my - h - 1)
            result[slp, L] = (
                result[slp, L].astype(f32) + slot_p[h].astype(f32)
            ).astype(bf16)
            rm.wait()
            slm = rows(my + h + 1)
            result[slm, R] = (
                result[slm, R].astype(f32) + slot_m[h].astype(f32)
            ).astype(bf16)

        pending = [None, None]

        def flush(idx, c, half):
            s = idx % 2
            if pending[s] is not None:
                pending[s].wait()
            fstage[s] = result[rows(c), half].astype(f32)
            d = pltpu.make_async_copy(
                fstage.at[s], out_ref.at[rows(c), half], flush_sems.at[s])
            d.start()
            pending[s] = d

        descs = []
        for h in range(N_DEV - 1):
            ap = pltpu.make_async_remote_copy(
                src_ref=result.at[rows(my + 1 - h), L],
                dst_ref=result.at[rows(my + 1 - h), L],
                send_sem=ag_send_p.at[h], recv_sem=ag_recv_p.at[h],
                device_id=(right,), device_id_type=MESH)
            am = pltpu.make_async_remote_copy(
                src_ref=result.at[rows(my - 1 + h), R],
                dst_ref=result.at[rows(my - 1 + h), R],
                send_sem=ag_send_m.at[h], recv_sem=ag_recv_m.at[h],
                device_id=(left,), device_id_type=MESH)
            if h > 0:
                descs[h - 1][0].wait_recv()
                descs[h - 1][1].wait_recv()
            ap.start()
            am.start()
            if h == 0:
                flush(0, my + 1, L)
                flush(1, my - 1, R)
            else:
                flush(2 * h, my - (h - 1), L)
                flush(2 * h + 1, my + (h - 1), R)
            descs.append((ap, am))

        descs[-1][0].wait_recv()
        descs[-1][1].wait_recv()
        flush(6, my - 2, L)
        flush(7, my + 2, R)
        pending[0].wait()
        pending[1].wait()
        for ap, am in descs:
            ap.wait_send()
            am.wait_send()

    return pl.pallas_call(
        body,
        out_shape=jax.ShapeDtypeStruct((m, n), jnp.float32),
        in_specs=[
            pl.BlockSpec(memory_space=pltpu.VMEM),
            pl.BlockSpec(memory_space=pltpu.VMEM),
        ],
        out_specs=pl.BlockSpec(memory_space=pl.ANY),
        scratch_shapes=[
            pltpu.VMEM((m, n), jnp.bfloat16),
            pltpu.VMEM((N_DEV - 1, mc, nh), jnp.bfloat16),
            pltpu.VMEM((N_DEV - 1, mc, nh), jnp.bfloat16),
            pltpu.VMEM((2, mc, nh), jnp.float32),
            pltpu.SemaphoreType.DMA((N_DEV - 1,)),
            pltpu.SemaphoreType.DMA((N_DEV - 1,)),
            pltpu.SemaphoreType.DMA((N_DEV - 1,)),
            pltpu.SemaphoreType.DMA((N_DEV - 1,)),
            pltpu.SemaphoreType.DMA((N_DEV - 1,)),
            pltpu.SemaphoreType.DMA((N_DEV - 1,)),
            pltpu.SemaphoreType.DMA((N_DEV - 1,)),
            pltpu.SemaphoreType.DMA((N_DEV - 1,)),
            pltpu.SemaphoreType.DMA((2,)),
        ],
        compiler_params=pltpu.CompilerParams(
            collective_id=0,
            vmem_limit_bytes=64 * 1024 * 1024,
        ),
    )(x, w_bf16)


# device time: 182557 ns/iter; 2.6959x vs baseline; 1.0623x over previous
import jax
import jax.numpy as jnp
from jax import lax
from jax.experimental import pallas as pl
from jax.experimental.pallas import tpu as pltpu

N_DEV = 4
MESH = pl.DeviceIdType.MESH


def kernel(x, w_mat):
    m, _ = x.shape
    _, n = w_mat.shape
    mc = m // N_DEV
    nh = n // 2
    hc = mc // 2

    w_bf16 = w_mat.astype(jnp.bfloat16)

    def body(x_ref, w_ref, out_ref, result, slot_p, slot_m, fstage,
             rs_send_p, rs_recv_p, rs_send_m, rs_recv_m,
             ag_send_p, ag_recv_p, ag_send_m, ag_recv_m,
             flush_sems):
        my = lax.axis_index("i")
        left = lax.rem(my + (N_DEV - 1), N_DEV)
        right = lax.rem(my + 1, N_DEV)

        barrier = pltpu.get_barrier_semaphore()
        for nbr in (left, right):
            pl.semaphore_signal(
                barrier, inc=1, device_id=(nbr,), device_id_type=MESH,
            )

        def rows(c):
            return pl.ds(lax.rem(c + 4 * N_DEV, N_DEV) * mc, mc)

        def rsub(c, j):
            return pl.ds(lax.rem(c + 4 * N_DEV, N_DEV) * mc + j * hc, hc)

        L = slice(0, nh)
        R = slice(nh, n)
        f32 = jnp.float32
        bf16 = jnp.bfloat16

        def mk(src, dst, ssem, rsem, dev):
            return pltpu.make_async_remote_copy(
                src_ref=src, dst_ref=dst, send_sem=ssem, recv_sem=rsem,
                device_id=(dev,), device_id_type=MESH)

        rs_p, rs_m, ag_p, ag_m = [], [], [], []
        for h in range(N_DEV - 1):
            rs_p.append([mk(result.at[rsub(my - h, j), L],
                            slot_p.at[h, slice(j * hc, (j + 1) * hc)],
                            rs_send_p.at[h, j], rs_recv_p.at[h, j], right)
                         for j in range(2)])
            rs_m.append([mk(result.at[rsub(my + h, j), R],
                            slot_m.at[h, slice(j * hc, (j + 1) * hc)],
                            rs_send_m.at[h, j], rs_recv_m.at[h, j], left)
                         for j in range(2)])
            ag_p.append([mk(result.at[rsub(my + 1 - h, j), L],
                            result.at[rsub(my + 1 - h, j), L],
                            ag_send_p.at[h, j], ag_recv_p.at[h, j], right)
                         for j in range(2)])
            ag_m.append([mk(result.at[rsub(my - 1 + h, j), R],
                            result.at[rsub(my - 1 + h, j), R],
                            ag_send_m.at[h, j], ag_recv_m.at[h, j], left)
                         for j in range(2)])

        def gemm(rsl, csl):
            result[rsl, csl] = jnp.dot(
                x_ref[rsl, :].astype(bf16), w_ref[:, csl],
                preferred_element_type=f32,
            ).astype(bf16)

        def acc(ring_slot, h, j, c, half):
            sl = rsub(c, j)
            ssl = slice(j * hc, (j + 1) * hc)
            result[sl, half] = (
                result[sl, half].astype(f32)
                + ring_slot[h, ssl].astype(f32)
            ).astype(bf16)

        gemm(rows(my), L)
        pl.semaphore_wait(barrier, 2)
        rs_p[0][0].start()
        rs_p[0][1].start()
        gemm(rows(my), R)
        rs_m[0][0].start()
        rs_m[0][1].start()
        gemm(rows(my - 1), slice(0, n))
        gemm(rows(my + 1), slice(0, n))
        gemm(rows(my + 2), slice(0, n))

        for h in range(N_DEV - 1):
            for j in range(2):
                rs_p[h][j].wait()
                acc(slot_p, h, j, my - h - 1, L)
                if h < N_DEV - 2:
                    rs_p[h + 1][j].start()
                else:
                    ag_p[0][j].start()
                rs_m[h][j].wait()
                acc(slot_m, h, j, my + h + 1, R)
                if h < N_DEV - 2:
                    rs_m[h + 1][j].start()
                else:
                    ag_m[0][j].start()

        pending = [None, None]

        def flush(idx, c, half):
            s = idx % 2
            if pending[s] is not None:
                pending[s].wait()
            fstage[s] = result[rows(c), half].astype(f32)
            d = pltpu.make_async_copy(
                fstage.at[s], out_ref.at[rows(c), half], flush_sems.at[s])
            d.start()
            pending[s] = d

        flush(0, my + 1, L)
        flush(1, my - 1, R)

        for h in range(N_DEV - 1):
            for j in range(2):
                ag_p[h][j].wait_recv()
                if h < N_DEV - 2:
                    ag_p[h + 1][j].start()
                ag_m[h][j].wait_recv()
                if h < N_DEV - 2:
                    ag_m[h + 1][j].start()
            flush(2 + 2 * h, my - h, L)
            flush(3 + 2 * h, my + h, R)

        pending[0].wait()
        pending[1].wait()
        for h in range(N_DEV - 1):
            for j in range(2):
                ag_p[h][j].wait_send()
                ag_m[h][j].wait_send()

    return pl.pallas_call(
        body,
        out_shape=jax.ShapeDtypeStruct((m, n), jnp.float32),
        in_specs=[
            pl.BlockSpec(memory_space=pltpu.VMEM),
            pl.BlockSpec(memory_space=pltpu.VMEM),
        ],
        out_specs=pl.BlockSpec(memory_space=pl.ANY),
        scratch_shapes=[
            pltpu.VMEM((m, n), jnp.bfloat16),
            pltpu.VMEM((N_DEV - 1, mc, nh), jnp.bfloat16),
            pltpu.VMEM((N_DEV - 1, mc, nh), jnp.bfloat16),
            pltpu.VMEM((2, mc, nh), jnp.float32),
            pltpu.SemaphoreType.DMA((N_DEV - 1, 2)),
            pltpu.SemaphoreType.DMA((N_DEV - 1, 2)),
            pltpu.SemaphoreType.DMA((N_DEV - 1, 2)),
            pltpu.SemaphoreType.DMA((N_DEV - 1, 2)),
            pltpu.SemaphoreType.DMA((N_DEV - 1, 2)),
            pltpu.SemaphoreType.DMA((N_DEV - 1, 2)),
            pltpu.SemaphoreType.DMA((N_DEV - 1, 2)),
            pltpu.SemaphoreType.DMA((N_DEV - 1, 2)),
            pltpu.SemaphoreType.DMA((2,)),
        ],
        compiler_params=pltpu.CompilerParams(
            collective_id=0,
            vmem_limit_bytes=64 * 1024 * 1024,
        ),
    )(x, w_bf16)


# device time: 182426 ns/iter; 2.6979x vs baseline; 1.0007x over previous
import jax
import jax.numpy as jnp
from jax import lax
from jax.experimental import pallas as pl
from jax.experimental.pallas import tpu as pltpu

N_DEV = 4
MESH = pl.DeviceIdType.MESH


def kernel(x, w_mat):
    m, _ = x.shape
    _, n = w_mat.shape
    mc = m // N_DEV
    nh = n // 2
    hc = mc // 2

    w_bf16 = w_mat.astype(jnp.bfloat16)

    def body(x_ref, w_ref, out_ref, res_l, res_r, slot_p, slot_m, fstage,
             rs_send_p, rs_recv_p, rs_send_m, rs_recv_m,
             ag_send_p, ag_recv_p, ag_send_m, ag_recv_m,
             flush_sems):
        my = lax.axis_index("i")
        left = lax.rem(my + (N_DEV - 1), N_DEV)
        right = lax.rem(my + 1, N_DEV)

        barrier = pltpu.get_barrier_semaphore()
        for nbr in (left, right):
            pl.semaphore_signal(
                barrier, inc=1, device_id=(nbr,), device_id_type=MESH,
            )

        def rows(c):
            return pl.ds(lax.rem(c + 4 * N_DEV, N_DEV) * mc, mc)

        def rsub(c, j):
            return pl.ds(lax.rem(c + 4 * N_DEV, N_DEV) * mc + j * hc, hc)

        f32 = jnp.float32
        bf16 = jnp.bfloat16

        def mk(src, dst, ssem, rsem, dev):
            return pltpu.make_async_remote_copy(
                src_ref=src, dst_ref=dst, send_sem=ssem, recv_sem=rsem,
                device_id=(dev,), device_id_type=MESH)

        rs_p, rs_m, ag_p, ag_m = [], [], [], []
        for h in range(N_DEV - 1):
            rs_p.append([mk(res_l.at[rsub(my - h, j), :],
                            slot_p.at[h, slice(j * hc, (j + 1) * hc)],
                            rs_send_p.at[h, j], rs_recv_p.at[h, j], right)
                         for j in range(2)])
            rs_m.append([mk(res_r.at[rsub(my + h, j), :],
                            slot_m.at[h, slice(j * hc, (j + 1) * hc)],
                            rs_send_m.at[h, j], rs_recv_m.at[h, j], left)
                         for j in range(2)])
            ag_p.append([mk(res_l.at[rsub(my + 1 - h, j), :],
                            res_l.at[rsub(my + 1 - h, j), :],
                            ag_send_p.at[h, j], ag_recv_p.at[h, j], right)
                         for j in range(2)])
            ag_m.append([mk(res_r.at[rsub(my - 1 + h, j), :],
                            res_r.at[rsub(my - 1 + h, j), :],
                            ag_send_m.at[h, j], ag_recv_m.at[h, j], left)
                         for j in range(2)])

        def gemm(res, rsl, csl):
            res[rsl, :] = jnp.dot(
                x_ref[rsl, :].astype(bf16), w_ref[:, csl],
                preferred_element_type=f32,
            ).astype(bf16)

        L = slice(0, nh)
        R = slice(nh, n)

        def acc(res, ring_slot, h, j, c):
            sl = rsub(c, j)
            ssl = slice(j * hc, (j + 1) * hc)
            res[sl, :] = (
                res[sl, :].astype(f32) + ring_slot[h, ssl].astype(f32)
            ).astype(bf16)

        gemm(res_l, rows(my), L)
        pl.semaphore_wait(barrier, 2)
        rs_p[0][0].start()
        rs_p[0][1].start()
        gemm(res_r, rows(my), R)
        rs_m[0][0].start()
        rs_m[0][1].start()
        for c in (my - 1, my + 1, my + 2):
            gemm(res_l, rows(c), L)
            gemm(res_r, rows(c), R)

        for h in range(N_DEV - 1):
            for j in range(2):
                rs_p[h][j].wait()
                acc(res_l, slot_p, h, j, my - h - 1)
                if h < N_DEV - 2:
                    rs_p[h + 1][j].start()
                else:
                    ag_p[0][j].start()
                rs_m[h][j].wait()
                acc(res_r, slot_m, h, j, my + h + 1)
                if h < N_DEV - 2:
                    rs_m[h + 1][j].start()
                else:
                    ag_m[0][j].start()

        pending = [None, None]

        def flush(idx, res, c, half):
            s = idx % 2
            if pending[s] is not None:
                pending[s].wait()
            fstage[s] = res[rows(c), :].astype(f32)
            d = pltpu.make_async_copy(
                fstage.at[s], out_ref.at[rows(c), half], flush_sems.at[s])
            d.start()
            pending[s] = d

        flush(0, res_l, my + 1, L)
        flush(1, res_r, my - 1, R)

        for h in range(N_DEV - 1):
            for j in range(2):
                ag_p[h][j].wait_recv()
                if h < N_DEV - 2:
                    ag_p[h + 1][j].start()
                ag_m[h][j].wait_recv()
                if h < N_DEV - 2:
                    ag_m[h + 1][j].start()
            flush(2 + 2 * h, res_l, my - h, L)
            flush(3 + 2 * h, res_r, my + h, R)

        pending[0].wait()
        pending[1].wait()
        for h in range(N_DEV - 1):
            for j in range(2):
                ag_p[h][j].wait_send()
                ag_m[h][j].wait_send()

    return pl.pallas_call(
        body,
        out_shape=jax.ShapeDtypeStruct((m, n), jnp.float32),
        in_specs=[
            pl.BlockSpec(memory_space=pltpu.VMEM),
            pl.BlockSpec(memory_space=pltpu.VMEM),
        ],
        out_specs=pl.BlockSpec(memory_space=pl.ANY),
        scratch_shapes=[
            pltpu.VMEM((m, nh), jnp.bfloat16),
            pltpu.VMEM((m, nh), jnp.bfloat16),
            pltpu.VMEM((N_DEV - 1, mc, nh), jnp.bfloat16),
            pltpu.VMEM((N_DEV - 1, mc, nh), jnp.bfloat16),
            pltpu.VMEM((2, mc, nh), jnp.float32),
            pltpu.SemaphoreType.DMA((N_DEV - 1, 2)),
            pltpu.SemaphoreType.DMA((N_DEV - 1, 2)),
            pltpu.SemaphoreType.DMA((N_DEV - 1, 2)),
            pltpu.SemaphoreType.DMA((N_DEV - 1, 2)),
            pltpu.SemaphoreType.DMA((N_DEV - 1, 2)),
            pltpu.SemaphoreType.DMA((N_DEV - 1, 2)),
            pltpu.SemaphoreType.DMA((N_DEV - 1, 2)),
            pltpu.SemaphoreType.DMA((N_DEV - 1, 2)),
            pltpu.SemaphoreType.DMA((2,)),
        ],
        compiler_params=pltpu.CompilerParams(
            collective_id=0,
            vmem_limit_bytes=64 * 1024 * 1024,
        ),
    )(x, w_bf16)


# device time: 177851 ns/iter; 2.7673x vs baseline; 1.0257x over previous
import jax
import jax.numpy as jnp
from jax import lax
from jax.experimental import pallas as pl
from jax.experimental.pallas import tpu as pltpu

N_DEV = 4
MESH = pl.DeviceIdType.MESH


def kernel(x, w_mat):
    m, k = x.shape
    _, n = w_mat.shape
    mc = m // N_DEV
    nh = n // 2
    hc = mc // 2

    w_bf16 = w_mat.astype(jnp.bfloat16)

    def body(x_ref, w_ref, out_ref, res_l, res_r, slot_p, slot_m, fstage,
             xbuf,
             rs_send_p, rs_recv_p, rs_send_m, rs_recv_m,
             ag_send_p, ag_recv_p, ag_send_m, ag_recv_m,
             flush_sems, x_sems):
        my = lax.axis_index("i")
        left = lax.rem(my + (N_DEV - 1), N_DEV)
        right = lax.rem(my + 1, N_DEV)

        barrier = pltpu.get_barrier_semaphore()
        for nbr in (left, right):
            pl.semaphore_signal(
                barrier, inc=1, device_id=(nbr,), device_id_type=MESH,
            )

        def rows(c):
            return pl.ds(lax.rem(c + 4 * N_DEV, N_DEV) * mc, mc)

        def rsub(c, j):
            return pl.ds(lax.rem(c + 4 * N_DEV, N_DEV) * mc + j * hc, hc)

        f32 = jnp.float32
        bf16 = jnp.bfloat16

        def mk(src, dst, ssem, rsem, dev):
            return pltpu.make_async_remote_copy(
                src_ref=src, dst_ref=dst, send_sem=ssem, recv_sem=rsem,
                device_id=(dev,), device_id_type=MESH)

        rs_p, rs_m, ag_p, ag_m = [], [], [], []
        for h in range(N_DEV - 1):
            rs_p.append([mk(res_l.at[rsub(my - h, j), :],
                            slot_p.at[h, slice(j * hc, (j + 1) * hc)],
                            rs_send_p.at[h, j], rs_recv_p.at[h, j], right)
                         for j in range(2)])
            rs_m.append([mk(res_r.at[rsub(my + h, j), :],
                            slot_m.at[h, slice(j * hc, (j + 1) * hc)],
                            rs_send_m.at[h, j], rs_recv_m.at[h, j], left)
                         for j in range(2)])
            ag_p.append([mk(res_l.at[rsub(my + 1 - h, j), :],
                            res_l.at[rsub(my + 1 - h, j), :],
                            ag_send_p.at[h, j], ag_recv_p.at[h, j], right)
                         for j in range(2)])
            ag_m.append([mk(res_r.at[rsub(my - 1 + h, j), :],
                            res_r.at[rsub(my - 1 + h, j), :],
                            ag_send_m.at[h, j], ag_recv_m.at[h, j], left)
                         for j in range(2)])

        def gemm(res, s, c, csl):
            res[rows(c), :] = jnp.dot(
                xbuf[s].astype(bf16), w_ref[:, csl],
                preferred_element_type=f32,
            ).astype(bf16)

        L = slice(0, nh)
        R = slice(nh, n)

        def acc(res, ring_slot, h, j, c):
            sl = rsub(c, j)
            ssl = slice(j * hc, (j + 1) * hc)
            res[sl, :] = res[sl, :] + ring_slot[h, ssl]

        def xload(s, c):
            d = pltpu.make_async_copy(
                x_ref.at[rows(c), :], xbuf.at[s], x_sems.at[s])
            d.start()
            return d

        ld = xload(0, my)
        ld.wait()
        gemm(res_l, 0, my, L)
        pl.semaphore_wait(barrier, 2)
        rs_p[0][0].start()
        rs_p[0][1].start()
        ld1 = xload(1, my - 1)
        gemm(res_r, 0, my, R)
        rs_m[0][0].start()
        rs_m[0][1].start()
        ld1.wait()
        ld2 = xload(0, my + 1)
        gemm(res_l, 1, my - 1, L)
        gemm(res_r, 1, my - 1, R)
        ld2.wait()
        ld3 = xload(1, my + 2)
        gemm(res_l, 0, my + 1, L)
        gemm(res_r, 0, my + 1, R)
        ld3.wait()
        gemm(res_l, 1, my + 2, L)
        gemm(res_r, 1, my + 2, R)

        for h in range(N_DEV - 1):
            for j in range(2):
                rs_p[h][j].wait()
                acc(res_l, slot_p, h, j, my - h - 1)
                if h < N_DEV - 2:
                    rs_p[h + 1][j].start()
                else:
                    ag_p[0][j].start()
                rs_m[h][j].wait()
                acc(res_r, slot_m, h, j, my + h + 1)
                if h < N_DEV - 2:
                    rs_m[h + 1][j].start()
                else:
                    ag_m[0][j].start()

        pending = [None, None]

        def flush(idx, res, c, half):
            s = idx % 2
            if pending[s] is not None:
                pending[s].wait()
            fstage[s] = res[rows(c), :].astype(f32)
            d = pltpu.make_async_copy(
                fstage.at[s], out_ref.at[rows(c), half], flush_sems.at[s])
            d.start()
            pending[s] = d

        def flush_sub(idx, res, c, j, half):
            s = idx % 2
            if pending[s] is not None:
                pending[s].wait()
            ssl = slice(j * hc, (j + 1) * hc)
            fstage[s, ssl] = res[rsub(c, j), :].astype(f32)
            d = pltpu.make_async_copy(
                fstage.at[s, ssl], out_ref.at[rsub(c, j), half],
                flush_sems.at[s])
            d.start()
            pending[s] = d

        flush(0, res_l, my + 1, L)
        flush(1, res_r, my - 1, R)

        for h in range(N_DEV - 1):
            last = h == N_DEV - 2
            for j in range(2):
                ag_p[h][j].wait_recv()
                if not last:
                    ag_p[h + 1][j].start()
                else:
                    flush_sub(2 * j, res_l, my - h, j, L)
                ag_m[h][j].wait_recv()
                if not last:
                    ag_m[h + 1][j].start()
                else:
                    flush_sub(2 * j + 1, res_r, my + h, j, R)
            if not last:
                flush(2 + 2 * h, res_l, my - h, L)
                flush(3 + 2 * h, res_r, my + h, R)

        pending[0].wait()
        pending[1].wait()
        for h in range(N_DEV - 1):
            for j in range(2):
                ag_p[h][j].wait_send()
                ag_m[h][j].wait_send()

    return pl.pallas_call(
        body,
        out_shape=jax.ShapeDtypeStruct((m, n), jnp.float32),
        in_specs=[
            pl.BlockSpec(memory_space=pl.ANY),
            pl.BlockSpec(memory_space=pltpu.VMEM),
        ],
        out_specs=pl.BlockSpec(memory_space=pl.ANY),
        scratch_shapes=[
            pltpu.VMEM((m, nh), jnp.bfloat16),
            pltpu.VMEM((m, nh), jnp.bfloat16),
            pltpu.VMEM((N_DEV - 1, mc, nh), jnp.bfloat16),
            pltpu.VMEM((N_DEV - 1, mc, nh), jnp.bfloat16),
            pltpu.VMEM((2, mc, nh), jnp.float32),
            pltpu.VMEM((2, mc, k), jnp.float32),
            pltpu.SemaphoreType.DMA((N_DEV - 1, 2)),
            pltpu.SemaphoreType.DMA((N_DEV - 1, 2)),
            pltpu.SemaphoreType.DMA((N_DEV - 1, 2)),
            pltpu.SemaphoreType.DMA((N_DEV - 1, 2)),
            pltpu.SemaphoreType.DMA((N_DEV - 1, 2)),
            pltpu.SemaphoreType.DMA((N_DEV - 1, 2)),
            pltpu.SemaphoreType.DMA((N_DEV - 1, 2)),
            pltpu.SemaphoreType.DMA((N_DEV - 1, 2)),
            pltpu.SemaphoreType.DMA((2,)),
            pltpu.SemaphoreType.DMA((2,)),
        ],
        compiler_params=pltpu.CompilerParams(
            collective_id=0,
            vmem_limit_bytes=64 * 1024 * 1024,
        ),
    )(x, w_bf16)


# device time: 173857 ns/iter; 2.8308x vs baseline; 1.0230x over previous
import jax
import jax.numpy as jnp
from jax import lax
from jax.experimental import pallas as pl
from jax.experimental.pallas import tpu as pltpu

N_DEV = 4
MESH = pl.DeviceIdType.MESH


def kernel(x, w_mat):
    m, k = x.shape
    _, n = w_mat.shape
    mc = m // N_DEV
    nh = n // 2
    hc = mc // 2

    def body(x_ref, w_ref, out_ref, slot_p, slot_m, xbuf, wb,
             rs_send_p, rs_recv_p, rs_send_m, rs_recv_m,
             ag_send_p, ag_recv_p, ag_send_m, ag_recv_m,
             x_sems):
        my = lax.axis_index("i")
        left = lax.rem(my + (N_DEV - 1), N_DEV)
        right = lax.rem(my + 1, N_DEV)

        barrier = pltpu.get_barrier_semaphore()
        for nbr in (left, right):
            pl.semaphore_signal(
                barrier, inc=1, device_id=(nbr,), device_id_type=MESH,
            )

        def rows(c):
            return pl.ds(lax.rem(c + 4 * N_DEV, N_DEV) * mc, mc)

        def rsub(c, j):
            return pl.ds(lax.rem(c + 4 * N_DEV, N_DEV) * mc + j * hc, hc)

        f32 = jnp.float32
        bf16 = jnp.bfloat16
        L = slice(0, nh)
        R = slice(nh, n)

        def mk(src, dst, ssem, rsem, dev):
            return pltpu.make_async_remote_copy(
                src_ref=src, dst_ref=dst, send_sem=ssem, recv_sem=rsem,
                device_id=(dev,), device_id_type=MESH)

        rs_p, rs_m, ag_p, ag_m = [], [], [], []
        for h in range(N_DEV - 1):
            rs_p.append([mk(out_ref.at[rsub(my - h, j), L],
                            slot_p.at[h, slice(j * hc, (j + 1) * hc)],
                            rs_send_p.at[h, j], rs_recv_p.at[h, j], right)
                         for j in range(2)])
            rs_m.append([mk(out_ref.at[rsub(my + h, j), R],
                            slot_m.at[h, slice(j * hc, (j + 1) * hc)],
                            rs_send_m.at[h, j], rs_recv_m.at[h, j], left)
                         for j in range(2)])
            ag_p.append([mk(out_ref.at[rsub(my + 1 - h, j), L],
                            out_ref.at[rsub(my + 1 - h, j), L],
                            ag_send_p.at[h, j], ag_recv_p.at[h, j], right)
                         for j in range(2)])
            ag_m.append([mk(out_ref.at[rsub(my - 1 + h, j), R],
                            out_ref.at[rsub(my - 1 + h, j), R],
                            ag_send_m.at[h, j], ag_recv_m.at[h, j], left)
                         for j in range(2)])

        def gemm(s, c, csl):
            out_ref[rows(c), csl] = jnp.dot(
                xbuf[s].astype(bf16), wb[:, csl],
                preferred_element_type=f32,
            ).astype(bf16)

        def acc(ring_slot, h, j, c, csl):
            sl = rsub(c, j)
            ssl = slice(j * hc, (j + 1) * hc)
            out_ref[sl, csl] = out_ref[sl, csl] + ring_slot[h, ssl]

        def xload(s, c):
            d = pltpu.make_async_copy(
                x_ref.at[rows(c), :], xbuf.at[s], x_sems.at[s])
            d.start()
            return d

        ld = xload(0, my)
        wb[...] = w_ref[...].astype(bf16)
        ld.wait()
        gemm(0, my, L)
        pl.semaphore_wait(barrier, 2)
        rs_p[0][0].start()
        rs_p[0][1].start()
        ld1 = xload(1, my - 1)
        gemm(0, my, R)
        rs_m[0][0].start()
        rs_m[0][1].start()
        ld1.wait()
        ld2 = xload(0, my + 1)
        gemm(1, my - 1, L)
        gemm(1, my - 1, R)
        ld2.wait()
        ld3 = xload(1, my + 2)
        gemm(0, my + 1, L)
        gemm(0, my + 1, R)
        ld3.wait()
        gemm(1, my + 2, L)
        gemm(1, my + 2, R)

        for h in range(N_DEV - 1):
            for j in range(2):
                rs_p[h][j].wait()
                acc(slot_p, h, j, my - h - 1, L)
                if h < N_DEV - 2:
                    rs_p[h + 1][j].start()
                else:
                    ag_p[0][j].start()
                rs_m[h][j].wait()
                acc(slot_m, h, j, my + h + 1, R)
                if h < N_DEV - 2:
                    rs_m[h + 1][j].start()
                else:
                    ag_m[0][j].start()

        for h in range(N_DEV - 1):
            for j in range(2):
                ag_p[h][j].wait_recv()
                if h < N_DEV - 2:
                    ag_p[h + 1][j].start()
                ag_m[h][j].wait_recv()
                if h < N_DEV - 2:
                    ag_m[h + 1][j].start()

        for h in range(N_DEV - 1):
            for j in range(2):
                ag_p[h][j].wait_send()
                ag_m[h][j].wait_send()

    out_bf16 = pl.pallas_call(
        body,
        out_shape=jax.ShapeDtypeStruct((m, n), jnp.bfloat16),
        in_specs=[
            pl.BlockSpec(memory_space=pl.ANY),
            pl.BlockSpec(memory_space=pltpu.VMEM),
        ],
        out_specs=pl.BlockSpec(memory_space=pltpu.VMEM),
        scratch_shapes=[
            pltpu.VMEM((N_DEV - 1, mc, nh), jnp.bfloat16),
            pltpu.VMEM((N_DEV - 1, mc, nh), jnp.bfloat16),
            pltpu.VMEM((2, mc, k), jnp.float32),
            pltpu.VMEM((k, n), jnp.bfloat16),
            pltpu.SemaphoreType.DMA((N_DEV - 1, 2)),
            pltpu.SemaphoreType.DMA((N_DEV - 1, 2)),
            pltpu.SemaphoreType.DMA((N_DEV - 1, 2)),
            pltpu.SemaphoreType.DMA((N_DEV - 1, 2)),
            pltpu.SemaphoreType.DMA((N_DEV - 1, 2)),
            pltpu.SemaphoreType.DMA((N_DEV - 1, 2)),
            pltpu.SemaphoreType.DMA((N_DEV - 1, 2)),
            pltpu.SemaphoreType.DMA((N_DEV - 1, 2)),
            pltpu.SemaphoreType.DMA((2,)),
        ],
        compiler_params=pltpu.CompilerParams(
            collective_id=0,
            vmem_limit_bytes=64 * 1024 * 1024,
        ),
    )(x, w_mat)
    return out_bf16.astype(jnp.float32)


# device time: 169137 ns/iter; 2.9098x vs baseline; 1.0279x over previous
import jax
import jax.numpy as jnp
from jax import lax
from jax.experimental import pallas as pl
from jax.experimental.pallas import tpu as pltpu

N_DEV = 4
MESH = pl.DeviceIdType.MESH


def kernel(x, w_mat):
    m, k = x.shape
    _, n = w_mat.shape
    mc = m // N_DEV
    nh = n // 2
    hc = mc // 2

    def body(x_ref, w_ref, out_ref, res, slot_p, slot_m, xbuf, wb,
             rs_send_p, rs_recv_p, rs_send_m, rs_recv_m,
             ag_send_p, ag_recv_p, ag_send_m, ag_recv_m,
             x_sems, st_sems):
        my = lax.axis_index("i")
        left = lax.rem(my + (N_DEV - 1), N_DEV)
        right = lax.rem(my + 1, N_DEV)

        barrier = pltpu.get_barrier_semaphore()
        for nbr in (left, right):
            pl.semaphore_signal(
                barrier, inc=1, device_id=(nbr,), device_id_type=MESH,
            )

        def rows(c):
            return pl.ds(lax.rem(c + 4 * N_DEV, N_DEV) * mc, mc)

        def rsub(c, j):
            return pl.ds(lax.rem(c + 4 * N_DEV, N_DEV) * mc + j * hc, hc)

        f32 = jnp.float32
        bf16 = jnp.bfloat16
        L = slice(0, nh)
        R = slice(nh, n)

        def mk(src, dst, ssem, rsem, dev):
            return pltpu.make_async_remote_copy(
                src_ref=src, dst_ref=dst, send_sem=ssem, recv_sem=rsem,
                device_id=(dev,), device_id_type=MESH)

        rs_p, rs_m, ag_p, ag_m = [], [], [], []
        for h in range(N_DEV - 1):
            rs_p.append([mk(res.at[rsub(my - h, j), L],
                            slot_p.at[h, slice(j * hc, (j + 1) * hc)],
                            rs_send_p.at[h, j], rs_recv_p.at[h, j], right)
                         for j in range(2)])
            rs_m.append([mk(res.at[rsub(my + h, j), R],
                            slot_m.at[h, slice(j * hc, (j + 1) * hc)],
                            rs_send_m.at[h, j], rs_recv_m.at[h, j], left)
                         for j in range(2)])
            ag_p.append([mk(res.at[rsub(my + 1 - h, j), L],
                            res.at[rsub(my + 1 - h, j), L],
                            ag_send_p.at[h, j], ag_recv_p.at[h, j], right)
                         for j in range(2)])
            ag_m.append([mk(res.at[rsub(my - 1 + h, j), R],
                            res.at[rsub(my - 1 + h, j), R],
                            ag_send_m.at[h, j], ag_recv_m.at[h, j], left)
                         for j in range(2)])

        def gemm(s, c, csl):
            res[rows(c), csl] = jnp.dot(
                xbuf[s].astype(bf16), wb[:, csl],
                preferred_element_type=f32,
            ).astype(bf16)

        def acc(ring_slot, h, j, c, csl):
            sl = rsub(c, j)
            ssl = slice(j * hc, (j + 1) * hc)
            res[sl, csl] = res[sl, csl] + ring_slot[h, ssl]

        def xload(s, c):
            d = pltpu.make_async_copy(
                x_ref.at[rows(c), :], xbuf.at[s], x_sems.at[s])
            d.start()
            return d

        st_pending = [None] * 4
        st_ctr = [0]

        def store(sl, csl):
            i = st_ctr[0] % 4
            st_ctr[0] += 1
            if st_pending[i] is not None:
                st_pending[i].wait()
            d = pltpu.make_async_copy(
                res.at[sl, csl], out_ref.at[sl, csl], st_sems.at[i])
            d.start()
            st_pending[i] = d

        ld = xload(0, my)
        wb[:, L] = w_ref[:, L].astype(bf16)
        ld.wait()
        gemm(0, my, L)
        pl.semaphore_wait(barrier, 2)
        rs_p[0][0].start()
        rs_p[0][1].start()
        ld1 = xload(1, my - 1)
        wb[:, R] = w_ref[:, R].astype(bf16)
        gemm(0, my, R)
        rs_m[0][0].start()
        rs_m[0][1].start()
        ld1.wait()
        ld2 = xload(0, my + 1)
        gemm(1, my - 1, L)
        gemm(1, my - 1, R)
        ld2.wait()
        ld3 = xload(1, my + 2)
        gemm(0, my + 1, L)
        gemm(0, my + 1, R)
        ld3.wait()
        gemm(1, my + 2, L)
        gemm(1, my + 2, R)

        for h in range(N_DEV - 1):
            for j in range(2):
                rs_p[h][j].wait()
                acc(slot_p, h, j, my - h - 1, L)
                if h < N_DEV - 2:
                    rs_p[h + 1][j].start()
                else:
                    ag_p[0][j].start()
                rs_m[h][j].wait()
                acc(slot_m, h, j, my + h + 1, R)
                if h < N_DEV - 2:
                    rs_m[h + 1][j].start()
                else:
                    ag_m[0][j].start()
                    if j == 1:
                        store(rows(my + 1), L)
                        store(rows(my - 1), R)

        for h in range(N_DEV - 1):
            for j in range(2):
                ag_p[h][j].wait_recv()
                if h < N_DEV - 2:
                    ag_p[h + 1][j].start()
                store(rsub(my - h, j), L)
                ag_m[h][j].wait_recv()
                if h < N_DEV - 2:
                    ag_m[h + 1][j].start()
                store(rsub(my + h, j), R)

        for d in st_pending:
            if d is not None:
                d.wait()
        for h in range(N_DEV - 1):
            for j in range(2):
                ag_p[h][j].wait_send()
                ag_m[h][j].wait_send()

    out_bf16 = pl.pallas_call(
        body,
        out_shape=jax.ShapeDtypeStruct((m, n), jnp.bfloat16),
        in_specs=[
            pl.BlockSpec(memory_space=pl.ANY),
            pl.BlockSpec(memory_space=pltpu.VMEM),
        ],
        out_specs=pl.BlockSpec(memory_space=pl.ANY),
        scratch_shapes=[
            pltpu.VMEM((m, n), jnp.bfloat16),
            pltpu.VMEM((N_DEV - 1, mc, nh), jnp.bfloat16),
            pltpu.VMEM((N_DEV - 1, mc, nh), jnp.bfloat16),
            pltpu.VMEM((2, mc, k), jnp.float32),
            pltpu.VMEM((k, n), jnp.bfloat16),
            pltpu.SemaphoreType.DMA((N_DEV - 1, 2)),
            pltpu.SemaphoreType.DMA((N_DEV - 1, 2)),
            pltpu.SemaphoreType.DMA((N_DEV - 1, 2)),
            pltpu.SemaphoreType.DMA((N_DEV - 1, 2)),
            pltpu.SemaphoreType.DMA((N_DEV - 1, 2)),
            pltpu.SemaphoreType.DMA((N_DEV - 1, 2)),
            pltpu.SemaphoreType.DMA((N_DEV - 1, 2)),
            pltpu.SemaphoreType.DMA((N_DEV - 1, 2)),
            pltpu.SemaphoreType.DMA((2,)),
            pltpu.SemaphoreType.DMA((4,)),
        ],
        compiler_params=pltpu.CompilerParams(
            collective_id=0,
            vmem_limit_bytes=64 * 1024 * 1024,
        ),
    )(x, w_mat)
    return out_bf16.astype(jnp.float32)
